# baseline (speedup 1.0000x reference)
import functools
import math

import jax
import jax.numpy as jnp
from jax import lax
from jax.experimental import pallas as pl
from jax.experimental.pallas import tpu as pltpu
from jax.experimental.pallas import tpu_sc as plsc

_R = 4
_D2MAX = 16
_H = 128
_W = 128
_NCH = 6
_TILES = 32
_ROWS_PER_TILE = (_NCH * _H) // _TILES
_STAGE_ROWS = _ROWS_PER_TILE + 2 * _R
_ZOFF = _STAGE_ROWS * _W
_HEAD = 10
_NVEC = _W // 16


def _sorted_offsets():
    offs = []
    for dy in range(-_R, _R + 1):
        for dx in range(-_R, _R + 1):
            if dy == 0 and dx == 0:
                continue
            d2 = dy * dy + dx * dx
            if d2 <= _D2MAX:
                offs.append((d2, dy * _W + dx, dy, dx))
    offs.sort()
    return [(dy, dx, 1.0 / math.sqrt(d2)) for (d2, _, dy, dx) in offs]


_OFFS = _sorted_offsets()

_MESH = plsc.VectorSubcoreMesh(core_axis_name="c", subcore_axis_name="s")


@functools.partial(
    pl.kernel,
    mesh=_MESH,
    out_type=jax.ShapeDtypeStruct((_NCH * _H * _W,), jnp.float32),
    scratch_types=[
        pltpu.VMEM((_STAGE_ROWS * _W + 16,), jnp.float32),
        pltpu.VMEM((_ROWS_PER_TILE * _W,), jnp.float32),
        pltpu.SMEM((len(_OFFS),), jnp.int32),
        pltpu.SMEM((len(_OFFS),), jnp.int32),
        pltpu.SMEM((len(_OFFS),), jnp.float32),
    ],
    compiler_params=pltpu.CompilerParams(needs_layout_passes=False),
)
def _sc_fill(img_hbm, out_hbm, stage, outv, dytab, dxtab, wtab):
    for j, (dy, dx, w) in enumerate(_OFFS):
        dytab[j] = jnp.int32(dy)
        dxtab[j] = jnp.int32(dx)
        wtab[j] = jnp.float32(w)
    wid = lax.axis_index("s") * 2 + lax.axis_index("c")
    g0 = wid * _ROWS_PER_TILE
    ch0 = g0 >> 7
    m0 = g0 & 127
    r_start = jnp.minimum(ch0 * _H + jnp.maximum(m0 - _R, 0),
                          _NCH * _H - _STAGE_ROWS)
    pltpu.sync_copy(img_hbm.at[pl.ds(r_start * _W, _STAGE_ROWS * _W)],
                    stage.at[pl.ds(0, _STAGE_ROWS * _W)])
    stage[pl.ds(_ZOFF, 16)] = jnp.zeros((16,), jnp.float32)
    lanes = lax.iota(jnp.int32, 16)
    zvec = jnp.full((16,), _ZOFF, jnp.int32)
    mleft = {dx: lanes < -dx for dx in range(-_R, 0)}
    mright = {dx: lanes >= 16 - dx for dx in range(1, _R + 1)}

    def cand_idx(cidx, dv, doffs, k, dx, mask_l, mask_r):
        idx = cidx + doffs[k]
        if dv == 0 and dx < 0:
            idx = jnp.where(mask_l[dx], zvec, idx)
        elif dv == _VPI - 1 and dx > 0:
            idx = jnp.where(mask_r[dx], zvec, idx)
        return idx

    def scan(state, cidx, dv, doffs, offs, mask_l, mask_r):
        pc, sw, swv = state
        for k, (dy, dx, w) in enumerate(offs):
            cand = plsc.load_gather(
                stage, [cand_idx(cidx, dv, doffs, k, dx, mask_l, mask_r)])
            fl = plsc.bitcast(cand, jnp.int32) != 0
            gate = fl & (pc < 3.0)
            wm = jnp.where(gate, jnp.float32(w), 0.0)
            sw = sw + wm
            swv = swv + wm * cand
            pc = pc + jnp.where(fl, 1.0, 0.0)
        return pc, sw, swv

    _VPI = 2
    _IPR = _NVEC // _VPI

    def row_body(i, carry):
        r = i >> 2
        vbase = (i & 3) * _VPI
        g = g0 + r
        ch = g >> 7
        dy_ok = {dy: ((g + dy) >> 7) == ch for dy in range(-_R, _R + 1)}
        doffs = [jnp.where(dy_ok[dy], jnp.int32(dy * _W + dx), jnp.int32(0))
                 for dy, dx, _ in _OFFS[:_HEAD]]
        base = (g - r_start) * _W
        is_first = vbase == 0
        is_last = vbase == _NVEC - _VPI
        mask_l = {dx: jnp.logical_and(m, is_first) for dx, m in mleft.items()}
        mask_r = {dx: jnp.logical_and(m, is_last) for dx, m in mright.items()}
        rowmin = None
        for dv in range(_VPI):
            cidx = base + (vbase + dv) * 16 + lanes
            center = plsc.load_gather(stage, [cidx])
            fc = plsc.bitcast(center, jnp.int32) != 0
            pc = jnp.where(fc, 3.0, 0.0)
            zero = jnp.zeros((16,), jnp.float32)
            pc, sw, swv = scan((pc, zero, zero), cidx, dv, doffs,
                               _OFFS[:_HEAD], mask_l, mask_r)
            fill = swv / jnp.maximum(sw, 1e-30)
            outv[pl.ds(r * _W + (vbase + dv) * 16, 16)] = jnp.where(fc, center, fill)
            rowmin = pc if rowmin is None else jnp.minimum(rowmin, pc)

        @pl.when(jnp.min(rowmin) < 3.0)
        def _redo():
            def redo_vec(v, vcarry):
                cidx = base + v * 16 + lanes
                colv = lanes + v * 16
                center = plsc.load_gather(stage, [cidx])
                fc = plsc.bitcast(center, jnp.int32) != 0

                def fb_step(j, st):
                    pc, sw, swv = st
                    dy = dytab[j]
                    dx = dxtab[j]
                    ok = ((g + dy) >> 7) == ch
                    doff = jnp.where(ok, dy * _W + dx, 0)
                    col = colv + dx
                    bad = (col < 0) | (col >= _W)
                    idx = jnp.where(bad, zvec, cidx + doff)
                    cand = plsc.load_gather(stage, [idx])
                    fl = plsc.bitcast(cand, jnp.int32) != 0
                    gate = fl & (pc < 3.0)
                    wm = jnp.where(gate, wtab[j], 0.0)
                    return (pc + jnp.where(fl, 1.0, 0.0), sw + wm, swv + wm * cand)

                zero = jnp.zeros((16,), jnp.float32)
                _, sw, swv = lax.fori_loop(
                    0, len(_OFFS), fb_step,
                    (jnp.where(fc, 3.0, 0.0), zero, zero))
                fill = swv / jnp.maximum(sw, 1e-30)
                outv[pl.ds(r * _W + v * 16, 16)] = jnp.where(fc, center, fill)
                return vcarry

            lax.fori_loop(vbase, vbase + _VPI, redo_vec, 0)

        return carry

    lax.fori_loop(0, _IPR * _ROWS_PER_TILE, row_body, 0)
    pltpu.sync_copy(outv, out_hbm.at[pl.ds(g0 * _W, _ROWS_PER_TILE * _W)])


def kernel(coded):
    b, c, h, w = coded.shape
    out = _sc_fill(coded.reshape(-1))
    return out.reshape(b, c, h, w)


# speedup vs baseline: 1.0013x; 1.0013x over previous
import functools
import math

import jax
import jax.numpy as jnp
from jax import lax
from jax.experimental import pallas as pl
from jax.experimental.pallas import tpu as pltpu
from jax.experimental.pallas import tpu_sc as plsc

_R = 4
_D2MAX = 16
_H = 128
_W = 128
_NCH = 6
_TILES = 32
_ROWS_PER_TILE = (_NCH * _H) // _TILES
_STAGE_ROWS = _ROWS_PER_TILE + 2 * _R
_ZOFF = _STAGE_ROWS * _W
_HEAD = 12
_NVEC = _W // 16


def _sorted_offsets():
    offs = []
    for dy in range(-_R, _R + 1):
        for dx in range(-_R, _R + 1):
            if dy == 0 and dx == 0:
                continue
            d2 = dy * dy + dx * dx
            if d2 <= _D2MAX:
                offs.append((d2, dy * _W + dx, dy, dx))
    offs.sort()
    return [(dy, dx, 1.0 / math.sqrt(d2)) for (d2, _, dy, dx) in offs]


_OFFS = _sorted_offsets()

_MESH = plsc.VectorSubcoreMesh(core_axis_name="c", subcore_axis_name="s")


@functools.partial(
    pl.kernel,
    mesh=_MESH,
    out_type=jax.ShapeDtypeStruct((_NCH * _H * _W,), jnp.float32),
    scratch_types=[
        pltpu.VMEM((_STAGE_ROWS * _W + 16,), jnp.float32),
        pltpu.VMEM((_ROWS_PER_TILE * _W,), jnp.float32),
        pltpu.SMEM((len(_OFFS),), jnp.int32),
        pltpu.SMEM((len(_OFFS),), jnp.int32),
        pltpu.SMEM((len(_OFFS),), jnp.float32),
    ],
    compiler_params=pltpu.CompilerParams(needs_layout_passes=False),
)
def _sc_fill(img_hbm, out_hbm, stage, outv, dytab, dxtab, wtab):
    for j, (dy, dx, w) in enumerate(_OFFS):
        dytab[j] = jnp.int32(dy)
        dxtab[j] = jnp.int32(dx)
        wtab[j] = jnp.float32(w)
    wid = lax.axis_index("s") * 2 + lax.axis_index("c")
    g0 = wid * _ROWS_PER_TILE
    ch0 = g0 >> 7
    m0 = g0 & 127
    r_start = jnp.minimum(ch0 * _H + jnp.maximum(m0 - _R, 0),
                          _NCH * _H - _STAGE_ROWS)
    pltpu.sync_copy(img_hbm.at[pl.ds(r_start * _W, _STAGE_ROWS * _W)],
                    stage.at[pl.ds(0, _STAGE_ROWS * _W)])
    stage[pl.ds(_ZOFF, 16)] = jnp.zeros((16,), jnp.float32)
    lanes = lax.iota(jnp.int32, 16)
    zvec = jnp.full((16,), _ZOFF, jnp.int32)
    mleft = {dx: lanes < -dx for dx in range(-_R, 0)}
    mright = {dx: lanes >= 16 - dx for dx in range(1, _R + 1)}

    def cand_idx(cidx, dv, doffs, k, dx, mask_l, mask_r):
        idx = cidx + doffs[k]
        if dv == 0 and dx < 0:
            idx = jnp.where(mask_l[dx], zvec, idx)
        elif dv == _VPI - 1 and dx > 0:
            idx = jnp.where(mask_r[dx], zvec, idx)
        return idx

    def scan(state, cidx, dv, doffs, offs, mask_l, mask_r):
        pc, sw, swv = state
        for k, (dy, dx, w) in enumerate(offs):
            cand = plsc.load_gather(
                stage, [cand_idx(cidx, dv, doffs, k, dx, mask_l, mask_r)])
            fl = plsc.bitcast(cand, jnp.int32) != 0
            gate = fl & (pc < 3.0)
            wm = jnp.where(gate, jnp.float32(w), 0.0)
            sw = sw + wm
            swv = swv + wm * cand
            pc = pc + jnp.where(fl, 1.0, 0.0)
        return pc, sw, swv

    _VPI = 2
    _IPR = _NVEC // _VPI

    def row_body(i, carry):
        r = i >> 2
        vbase = (i & 3) * _VPI
        g = g0 + r
        ch = g >> 7
        dy_ok = {dy: ((g + dy) >> 7) == ch for dy in range(-_R, _R + 1)}
        doffs = [jnp.where(dy_ok[dy], jnp.int32(dy * _W + dx), jnp.int32(0))
                 for dy, dx, _ in _OFFS[:_HEAD]]
        base = (g - r_start) * _W
        is_first = vbase == 0
        is_last = vbase == _NVEC - _VPI
        mask_l = {dx: jnp.logical_and(m, is_first) for dx, m in mleft.items()}
        mask_r = {dx: jnp.logical_and(m, is_last) for dx, m in mright.items()}
        rowmin = None
        for dv in range(_VPI):
            cidx = base + (vbase + dv) * 16 + lanes
            center = plsc.load_gather(stage, [cidx])
            fc = plsc.bitcast(center, jnp.int32) != 0
            pc = jnp.where(fc, 3.0, 0.0)
            zero = jnp.zeros((16,), jnp.float32)
            pc, sw, swv = scan((pc, zero, zero), cidx, dv, doffs,
                               _OFFS[:_HEAD], mask_l, mask_r)
            fill = swv / jnp.maximum(sw, 1e-30)
            outv[pl.ds(r * _W + (vbase + dv) * 16, 16)] = jnp.where(fc, center, fill)
            rowmin = pc if rowmin is None else jnp.minimum(rowmin, pc)

        @pl.when(jnp.min(rowmin) < 3.0)
        def _redo():
            def redo_vec(v, vcarry):
                cidx = base + v * 16 + lanes
                colv = lanes + v * 16
                center = plsc.load_gather(stage, [cidx])
                fc = plsc.bitcast(center, jnp.int32) != 0

                def fb_step(j, st):
                    pc, sw, swv = st
                    dy = dytab[j]
                    dx = dxtab[j]
                    ok = ((g + dy) >> 7) == ch
                    doff = jnp.where(ok, dy * _W + dx, 0)
                    col = colv + dx
                    bad = (col < 0) | (col >= _W)
                    idx = jnp.where(bad, zvec, cidx + doff)
                    cand = plsc.load_gather(stage, [idx])
                    fl = plsc.bitcast(cand, jnp.int32) != 0
                    gate = fl & (pc < 3.0)
                    wm = jnp.where(gate, wtab[j], 0.0)
                    return (pc + jnp.where(fl, 1.0, 0.0), sw + wm, swv + wm * cand)

                zero = jnp.zeros((16,), jnp.float32)
                _, sw, swv = lax.fori_loop(
                    0, len(_OFFS), fb_step,
                    (jnp.where(fc, 3.0, 0.0), zero, zero))
                fill = swv / jnp.maximum(sw, 1e-30)
                outv[pl.ds(r * _W + v * 16, 16)] = jnp.where(fc, center, fill)
                return vcarry

            lax.fori_loop(vbase, vbase + _VPI, redo_vec, 0)

        return carry

    lax.fori_loop(0, _IPR * _ROWS_PER_TILE, row_body, 0)
    pltpu.sync_copy(outv, out_hbm.at[pl.ds(g0 * _W, _ROWS_PER_TILE * _W)])


def kernel(coded):
    b, c, h, w = coded.shape
    out = _sc_fill(coded.reshape(-1))
    return out.reshape(b, c, h, w)
